# Initial kernel scaffold; baseline (speedup 1.0000x reference)
import jax
import jax.numpy as jnp
from jax.experimental import pallas as pl


def kernel(x, edge_index, edge_label_index, W1, b1, W2, b2):
    raise NotImplementedError("write your pallas kernel here")


# speedup vs baseline: 2.8412x; 2.8412x over previous
import dataclasses
import functools

import jax
import jax.numpy as jnp
from jax import lax
from jax.experimental import pallas as pl
from jax.experimental.pallas import tpu as pltpu
from jax.experimental.pallas import tpu_sc as plsc

_F32 = jnp.float32
_NTILES = 32
_BLK = 128


def _mesh():
    return plsc.VectorSubcoreMesh(core_axis_name="c", subcore_axis_name="s")


def _sc_hist(col2d, npad):
    eb = col2d.shape[0] // _NTILES
    cp = pltpu.CompilerParams()
    if "needs_layout_passes" in pltpu.CompilerParams.__dataclass_fields__:
        cp = dataclasses.replace(cp, needs_layout_passes=False)

    @functools.partial(
        pl.kernel,
        out_type=jax.ShapeDtypeStruct((_NTILES, npad), _F32),
        mesh=_mesh(),
        compiler_params=cp,
        scratch_types=[
            pltpu.VMEM((eb, _BLK), jnp.int32),
            pltpu.VMEM((npad,), _F32),
        ],
    )
    def k(col_hbm, out_hbm, idx, hist):
        c = lax.axis_index("c")
        s = lax.axis_index("s")
        wid = c * 16 + s
        pltpu.sync_copy(col_hbm.at[pl.ds(wid * eb, eb)], idx)

        @pl.loop(0, npad, step=16)
        def _(i):
            hist[pl.ds(i, 16)] = jnp.zeros((16,), _F32)

        ones = jnp.ones((16,), _F32)

        @pl.loop(0, eb)
        def _(j):
            @pl.loop(0, _BLK, step=16)
            def _(t):
                plsc.addupdate_scatter(hist, [idx[j, pl.ds(t, 16)]], ones)

        pltpu.sync_copy(hist, out_hbm.at[wid])

    return k(col2d)


def _sc_agg(y, row2d, col2d, zeros_d, npad, d):
    eb = row2d.shape[0] // _NTILES
    rows = npad // 16

    @functools.partial(
        pl.kernel,
        out_type=jax.ShapeDtypeStruct((2, npad, d), _F32),
        mesh=_mesh(),
        scratch_types=[
            pltpu.VMEM((eb, _BLK), jnp.int32),
            pltpu.VMEM((eb, _BLK), jnp.int32),
            pltpu.VMEM((_BLK, d), _F32),
            pltpu.VMEM_SHARED((npad, d), _F32),
            pltpu.SemaphoreType.DMA,
        ],
    )
    def k(y_hbm, r_hbm, c_hbm, z_hbm, out_hbm, ridx, cidx, buf, acc, sem):
        c = lax.axis_index("c")
        s = lax.axis_index("s")
        wid = c * 16 + s
        pltpu.sync_copy(z_hbm.at[pl.ds(s * rows, rows)],
                        acc.at[pl.ds(s * rows, rows)])
        pltpu.sync_copy(r_hbm.at[pl.ds(wid * eb, eb)], ridx)
        pltpu.sync_copy(c_hbm.at[pl.ds(wid * eb, eb)], cidx)
        plsc.subcore_barrier()

        @pl.loop(0, eb)
        def _(j):
            pltpu.async_copy(y_hbm.at[ridx.at[j]], buf, sem).wait()
            pltpu.sync_copy(buf, acc.at[cidx.at[j]], add=True)

        plsc.subcore_barrier()
        pltpu.sync_copy(acc.at[pl.ds(s * rows, rows)],
                        out_hbm.at[c, pl.ds(s * rows, rows)])

    return k(y, row2d, col2d, zeros_d)


def _sc_decode(z, src2d, dst2d, d):
    lb = src2d.shape[0] // _NTILES
    lpad = src2d.shape[0] * _BLK // _NTILES * _NTILES
    nj = d // 16

    @functools.partial(
        pl.kernel,
        out_type=jax.ShapeDtypeStruct((lpad, 16), _F32),
        mesh=_mesh(),
        scratch_types=[
            pltpu.VMEM((lb, _BLK), jnp.int32),
            pltpu.VMEM((lb, _BLK), jnp.int32),
            pltpu.VMEM((_BLK, d), _F32),
            pltpu.VMEM((_BLK, d), _F32),
            pltpu.VMEM((_BLK, 16), _F32),
            pltpu.SemaphoreType.DMA,
            pltpu.SemaphoreType.DMA,
        ],
    )
    def k(z_hbm, s_hbm, d_hbm, out_hbm, sidx, didx, bufa, bufb, pbuf,
          sema, semb):
        c = lax.axis_index("c")
        s = lax.axis_index("s")
        wid = c * 16 + s
        pltpu.sync_copy(s_hbm.at[pl.ds(wid * lb, lb)], sidx)
        pltpu.sync_copy(d_hbm.at[pl.ds(wid * lb, lb)], didx)

        @pl.loop(0, lb)
        def _(i):
            cpa = pltpu.async_copy(z_hbm.at[sidx.at[i]], bufa, sema)
            cpb = pltpu.async_copy(z_hbm.at[didx.at[i]], bufb, semb)
            cpa.wait()
            cpb.wait()

            @pl.loop(0, _BLK)
            def _(e):
                a = (bufa[pl.ds(e, 1), pl.ds(0, 16)]
                     * bufb[pl.ds(e, 1), pl.ds(0, 16)])
                for j in range(1, nj):
                    a = a + (bufa[pl.ds(e, 1), pl.ds(16 * j, 16)]
                             * bufb[pl.ds(e, 1), pl.ds(16 * j, 16)])
                pbuf[pl.ds(e, 1), pl.ds(0, 16)] = a

            pltpu.sync_copy(pbuf, out_hbm.at[pl.ds((wid * lb + i) * _BLK,
                                                   _BLK)])

    return k(z, src2d, dst2d)


_HIGH = jax.lax.Precision.HIGHEST


def _dot(a, b):
    return lax.dot_general(a, b, (((1,), (0,)), ((), ())),
                           precision=_HIGH, preferred_element_type=_F32)


def _tc_matmul(x, w):
    def body(x_ref, w_ref, o_ref):
        o_ref[...] = _dot(x_ref[...], w_ref[...])
    return pl.pallas_call(
        body, out_shape=jax.ShapeDtypeStruct((x.shape[0], w.shape[1]), _F32),
    )(x, w)


def _dis(cnt_ref):
    deg = 1.0 + jnp.sum(cnt_ref[...], axis=1, keepdims=True)
    return 1.0 / jnp.sqrt(deg)


def _tc_prep(xw, cnt_t):
    def body(xw_ref, cnt_ref, o_ref):
        o_ref[...] = xw_ref[...] * _dis(cnt_ref)
    return pl.pallas_call(
        body, out_shape=jax.ShapeDtypeStruct(xw.shape, _F32),
    )(xw, cnt_t)


def _tc_layer_mid(a0, a1, y, cnt_t, b, w2):
    def body(a0_ref, a1_ref, y_ref, cnt_ref, b_ref, w_ref, o_ref):
        dis = _dis(cnt_ref)
        z = jnp.maximum(
            dis * (a0_ref[...] + a1_ref[...] + y_ref[...]) + b_ref[...], 0.0)
        o_ref[...] = _dot(z, w_ref[...]) * dis
    return pl.pallas_call(
        body, out_shape=jax.ShapeDtypeStruct(y.shape, _F32),
    )(a0, a1, y, cnt_t, b, w2)


def _tc_layer_last(a0, a1, y, cnt_t, b):
    def body(a0_ref, a1_ref, y_ref, cnt_ref, b_ref, o_ref):
        o_ref[...] = jnp.maximum(
            _dis(cnt_ref) * (a0_ref[...] + a1_ref[...] + y_ref[...])
            + b_ref[...], 0.0)
    return pl.pallas_call(
        body, out_shape=jax.ShapeDtypeStruct(y.shape, _F32),
    )(a0, a1, y, cnt_t, b)


def _tc_reduce(p):
    blk = 2048
    def body(p_ref, o_ref):
        o_ref[...] = jnp.sum(p_ref[...], axis=1, keepdims=True)
    return pl.pallas_call(
        body,
        grid=(p.shape[0] // blk,),
        in_specs=[pl.BlockSpec((blk, 16), lambda i: (i, 0))],
        out_specs=pl.BlockSpec((blk, 1), lambda i: (i, 0)),
        out_shape=jax.ShapeDtypeStruct((p.shape[0], 1), _F32),
    )(p)


def _pad_idx(a, pad_val, unit):
    n = a.shape[0]
    npad = -(-n // unit) * unit
    return jnp.concatenate(
        [a, jnp.full((npad - n,), pad_val, jnp.int32)]).reshape(-1, _BLK)


def kernel(x, edge_index, edge_label_index, W1, b1, W2, b2):
    n, d = x.shape
    npad = -(-(n + 16) // 2048) * 2048

    row = edge_index[0].astype(jnp.int32)
    col = edge_index[1].astype(jnp.int32)
    src = edge_label_index[0].astype(jnp.int32)
    dst = edge_label_index[1].astype(jnp.int32)
    nlabel = src.shape[0]

    unit = _NTILES * _BLK * 8
    row2d = _pad_idx(row, n, unit)
    col2d = _pad_idx(col, n + 8, unit)
    src2d = _pad_idx(src, 0, unit)
    dst2d = _pad_idx(dst, 0, unit)

    x_p = jnp.pad(x, ((0, npad - n), (0, 0)))
    zeros_d = jnp.zeros((npad, d), _F32)
    b1r = b1.reshape(1, d)
    b2r = b2.reshape(1, d)

    cnt = _sc_hist(col2d, npad)
    xw1 = _tc_matmul(x_p, W1)
    cnt_t = cnt.T

    y1 = _tc_prep(xw1, cnt_t)
    ag1 = _sc_agg(y1, row2d, col2d, zeros_d, npad, d)
    y2 = _tc_layer_mid(ag1[0], ag1[1], y1, cnt_t, b1r, W2)
    ag2 = _sc_agg(y2, row2d, col2d, zeros_d, npad, d)
    z2 = _tc_layer_last(ag2[0], ag2[1], y2, cnt_t, b2r)

    p = _sc_decode(z2, src2d, dst2d, d)
    scores = _tc_reduce(p)
    return scores[:nlabel, 0]


# speedup vs baseline: 2.9213x; 1.0282x over previous
import dataclasses
import functools

import jax
import jax.numpy as jnp
from jax import lax
from jax.experimental import pallas as pl
from jax.experimental.pallas import tpu as pltpu
from jax.experimental.pallas import tpu_sc as plsc

_F32 = jnp.float32
_NTILES = 32
_BLK = 128


def _mesh():
    return plsc.VectorSubcoreMesh(core_axis_name="c", subcore_axis_name="s")


def _sc_hist(col2d, npad):
    eb = col2d.shape[0] // _NTILES
    cp = pltpu.CompilerParams()
    if "needs_layout_passes" in pltpu.CompilerParams.__dataclass_fields__:
        cp = dataclasses.replace(cp, needs_layout_passes=False)

    @functools.partial(
        pl.kernel,
        out_type=jax.ShapeDtypeStruct((_NTILES, npad), _F32),
        mesh=_mesh(),
        compiler_params=cp,
        scratch_types=[
            pltpu.VMEM((eb, _BLK), jnp.int32),
            pltpu.VMEM((npad,), _F32),
        ],
    )
    def k(col_hbm, out_hbm, idx, hist):
        c = lax.axis_index("c")
        s = lax.axis_index("s")
        wid = c * 16 + s
        pltpu.sync_copy(col_hbm.at[pl.ds(wid * eb, eb)], idx)

        @pl.loop(0, npad, step=16)
        def _(i):
            hist[pl.ds(i, 16)] = jnp.zeros((16,), _F32)

        ones = jnp.ones((16,), _F32)

        @pl.loop(0, eb)
        def _(j):
            @pl.loop(0, _BLK, step=16)
            def _(t):
                plsc.addupdate_scatter(hist, [idx[j, pl.ds(t, 16)]], ones)

        pltpu.sync_copy(hist, out_hbm.at[wid])

    return k(col2d)


def _sc_agg(y, row2d, col2d, zeros_d, npad, d):
    eb = row2d.shape[0] // _NTILES
    rows = npad // 16
    ch = 16
    nch = eb // ch

    @functools.partial(
        pl.kernel,
        out_type=jax.ShapeDtypeStruct((2, npad, d), _F32),
        mesh=_mesh(),
        scratch_types=[
            pltpu.VMEM((ch, _BLK), jnp.int32),
            pltpu.VMEM((ch, _BLK), jnp.int32),
            pltpu.VMEM((_BLK, d), _F32),
            pltpu.VMEM((_BLK, d), _F32),
            pltpu.VMEM_SHARED((npad, d), _F32),
            pltpu.SemaphoreType.DMA,
            pltpu.SemaphoreType.DMA,
            pltpu.SemaphoreType.DMA,
            pltpu.SemaphoreType.DMA,
        ],
    )
    def k(y_hbm, r_hbm, c_hbm, z_hbm, out_hbm, ridx, cidx,
          buf0, buf1, acc, sg0, sg1, ss0, ss1):
        c = lax.axis_index("c")
        s = lax.axis_index("s")
        wid = c * 16 + s
        buf = (buf0, buf1)
        sg = (sg0, sg1)
        ss = (ss0, ss1)
        pltpu.sync_copy(z_hbm.at[pl.ds(s * rows, rows)],
                        acc.at[pl.ds(s * rows, rows)])
        plsc.subcore_barrier()

        @pl.loop(0, nch)
        def _(cc):
            base = wid * eb + cc * ch
            pltpu.sync_copy(r_hbm.at[pl.ds(base, ch)], ridx)
            pltpu.sync_copy(c_hbm.at[pl.ds(base, ch)], cidx)

            for b in range(2):
                pltpu.async_copy(y_hbm.at[ridx.at[b]], buf[b], sg[b])

            @pl.loop(0, ch - 2, step=2)
            def _(j):
                for b in range(2):
                    jb = j + b
                    pltpu.make_async_copy(y_hbm.at[ridx.at[jb]], buf[b],
                                          sg[b]).wait()
                    pltpu.async_copy(buf[b], acc.at[cidx.at[jb]], ss[b],
                                     add=True)
                for b in range(2):
                    jb = j + b
                    pltpu.make_async_copy(buf[b], acc.at[cidx.at[jb]],
                                          ss[b]).wait()
                    pltpu.async_copy(y_hbm.at[ridx.at[jb + 2]], buf[b],
                                     sg[b])

            for b in range(2):
                jb = ch - 2 + b
                pltpu.make_async_copy(y_hbm.at[ridx.at[jb]], buf[b],
                                      sg[b]).wait()
                pltpu.async_copy(buf[b], acc.at[cidx.at[jb]], ss[b],
                                 add=True)
            for b in range(2):
                jb = ch - 2 + b
                pltpu.make_async_copy(buf[b], acc.at[cidx.at[jb]],
                                      ss[b]).wait()

        plsc.subcore_barrier()
        pltpu.sync_copy(acc.at[pl.ds(s * rows, rows)],
                        out_hbm.at[c, pl.ds(s * rows, rows)])

    return k(y, row2d, col2d, zeros_d)


def _sc_decode(z, src2d, dst2d, d):
    lb = src2d.shape[0] // _NTILES
    lpad = src2d.size

    @functools.partial(
        pl.kernel,
        out_type=(jax.ShapeDtypeStruct((lpad, d), _F32),
                  jax.ShapeDtypeStruct((lpad, d), _F32)),
        mesh=_mesh(),
        scratch_types=[
            pltpu.VMEM((lb, _BLK), jnp.int32),
            pltpu.VMEM((lb, _BLK), jnp.int32),
            pltpu.VMEM((_BLK, d), _F32),
            pltpu.VMEM((_BLK, d), _F32),
            pltpu.VMEM((_BLK, d), _F32),
            pltpu.VMEM((_BLK, d), _F32),
            pltpu.SemaphoreType.DMA,
            pltpu.SemaphoreType.DMA,
            pltpu.SemaphoreType.DMA,
            pltpu.SemaphoreType.DMA,
            pltpu.SemaphoreType.DMA,
            pltpu.SemaphoreType.DMA,
            pltpu.SemaphoreType.DMA,
            pltpu.SemaphoreType.DMA,
        ],
    )
    def k(z_hbm, s_hbm, d_hbm, za_hbm, zb_hbm, sidx, didx,
          bufa0, bufa1, bufb0, bufb1,
          sga0, sga1, sgb0, sgb1, swa0, swa1, swb0, swb1):
        c = lax.axis_index("c")
        s = lax.axis_index("s")
        wid = c * 16 + s
        bufa = (bufa0, bufa1)
        bufb = (bufb0, bufb1)
        sga = (sga0, sga1)
        sgb = (sgb0, sgb1)
        swa = (swa0, swa1)
        swb = (swb0, swb1)
        pltpu.sync_copy(s_hbm.at[pl.ds(wid * lb, lb)], sidx)
        pltpu.sync_copy(d_hbm.at[pl.ds(wid * lb, lb)], didx)

        for b in range(2):
            pltpu.async_copy(z_hbm.at[sidx.at[b]], bufa[b], sga[b])
            pltpu.async_copy(z_hbm.at[didx.at[b]], bufb[b], sgb[b])

        def flush(b, ib):
            out = pl.ds((wid * lb + ib) * _BLK, _BLK)
            pltpu.make_async_copy(z_hbm.at[sidx.at[ib]], bufa[b],
                                  sga[b]).wait()
            pltpu.async_copy(bufa[b], za_hbm.at[out], swa[b])
            pltpu.make_async_copy(z_hbm.at[didx.at[ib]], bufb[b],
                                  sgb[b]).wait()
            pltpu.async_copy(bufb[b], zb_hbm.at[out], swb[b])

        def wait_wb(b, ib):
            out = pl.ds((wid * lb + ib) * _BLK, _BLK)
            pltpu.make_async_copy(bufa[b], za_hbm.at[out], swa[b]).wait()
            pltpu.make_async_copy(bufb[b], zb_hbm.at[out], swb[b]).wait()

        @pl.loop(0, lb - 2, step=2)
        def _(i):
            for b in range(2):
                ib = i + b
                flush(b, ib)
                wait_wb(b, ib)
                pltpu.async_copy(z_hbm.at[sidx.at[ib + 2]], bufa[b], sga[b])
                pltpu.async_copy(z_hbm.at[didx.at[ib + 2]], bufb[b], sgb[b])

        for b in range(2):
            flush(b, lb - 2 + b)
        for b in range(2):
            wait_wb(b, lb - 2 + b)

    return k(z, src2d, dst2d)


_HIGH = jax.lax.Precision.HIGHEST


def _dot(a, b):
    return lax.dot_general(a, b, (((1,), (0,)), ((), ())),
                           precision=_HIGH, preferred_element_type=_F32)


def _tc_matmul(x, w):
    def body(x_ref, w_ref, o_ref):
        o_ref[...] = _dot(x_ref[...], w_ref[...])
    return pl.pallas_call(
        body, out_shape=jax.ShapeDtypeStruct((x.shape[0], w.shape[1]), _F32),
    )(x, w)


def _dis(cnt_ref):
    deg = 1.0 + jnp.sum(cnt_ref[...], axis=1, keepdims=True)
    return 1.0 / jnp.sqrt(deg)


def _tc_prep(xw, cnt_t):
    def body(xw_ref, cnt_ref, o_ref):
        o_ref[...] = xw_ref[...] * _dis(cnt_ref)
    return pl.pallas_call(
        body, out_shape=jax.ShapeDtypeStruct(xw.shape, _F32),
    )(xw, cnt_t)


def _tc_layer_mid(a0, a1, y, cnt_t, b, w2):
    def body(a0_ref, a1_ref, y_ref, cnt_ref, b_ref, w_ref, o_ref):
        dis = _dis(cnt_ref)
        z = jnp.maximum(
            dis * (a0_ref[...] + a1_ref[...] + y_ref[...]) + b_ref[...], 0.0)
        o_ref[...] = _dot(z, w_ref[...]) * dis
    return pl.pallas_call(
        body, out_shape=jax.ShapeDtypeStruct(y.shape, _F32),
    )(a0, a1, y, cnt_t, b, w2)


def _tc_layer_last(a0, a1, y, cnt_t, b):
    def body(a0_ref, a1_ref, y_ref, cnt_ref, b_ref, o_ref):
        o_ref[...] = jnp.maximum(
            _dis(cnt_ref) * (a0_ref[...] + a1_ref[...] + y_ref[...])
            + b_ref[...], 0.0)
    return pl.pallas_call(
        body, out_shape=jax.ShapeDtypeStruct(y.shape, _F32),
    )(a0, a1, y, cnt_t, b)


def _tc_dotred(za, zb):
    blk = 4096
    d = za.shape[1]
    def body(a_ref, b_ref, o_ref):
        o_ref[...] = jnp.sum(a_ref[...] * b_ref[...], axis=1, keepdims=True)
    return pl.pallas_call(
        body,
        grid=(za.shape[0] // blk,),
        in_specs=[pl.BlockSpec((blk, d), lambda i: (i, 0)),
                  pl.BlockSpec((blk, d), lambda i: (i, 0))],
        out_specs=pl.BlockSpec((blk, 1), lambda i: (i, 0)),
        out_shape=jax.ShapeDtypeStruct((za.shape[0], 1), _F32),
    )(za, zb)


def _pad_idx(a, pad_val, unit):
    n = a.shape[0]
    npad = -(-n // unit) * unit
    return jnp.concatenate(
        [a, jnp.full((npad - n,), pad_val, jnp.int32)]).reshape(-1, _BLK)


def kernel(x, edge_index, edge_label_index, W1, b1, W2, b2):
    n, d = x.shape
    npad = -(-(n + 16) // 2048) * 2048

    row = edge_index[0].astype(jnp.int32)
    col = edge_index[1].astype(jnp.int32)
    src = edge_label_index[0].astype(jnp.int32)
    dst = edge_label_index[1].astype(jnp.int32)
    nlabel = src.shape[0]

    unit = _NTILES * _BLK * 8
    row2d = _pad_idx(row, n, unit)
    col2d = _pad_idx(col, n + 8, unit)
    src2d = _pad_idx(src, 0, unit)
    dst2d = _pad_idx(dst, 0, unit)

    x_p = jnp.pad(x, ((0, npad - n), (0, 0)))
    zeros_d = jnp.zeros((npad, d), _F32)
    b1r = b1.reshape(1, d)
    b2r = b2.reshape(1, d)

    cnt = _sc_hist(col2d, npad)
    xw1 = _tc_matmul(x_p, W1)
    cnt_t = cnt.T

    y1 = _tc_prep(xw1, cnt_t)
    ag1 = _sc_agg(y1, row2d, col2d, zeros_d, npad, d)
    y2 = _tc_layer_mid(ag1[0], ag1[1], y1, cnt_t, b1r, W2)
    ag2 = _sc_agg(y2, row2d, col2d, zeros_d, npad, d)
    z2 = _tc_layer_last(ag2[0], ag2[1], y2, cnt_t, b2r)

    za, zb = _sc_decode(z2, src2d, dst2d, d)
    scores = _tc_dotred(za, zb)
    return scores[:nlabel, 0]


# speedup vs baseline: 3.2836x; 1.1240x over previous
import dataclasses
import functools

import jax
import jax.numpy as jnp
from jax import lax
from jax.experimental import pallas as pl
from jax.experimental.pallas import tpu as pltpu
from jax.experimental.pallas import tpu_sc as plsc

_F32 = jnp.float32
_NTILES = 32
_NSUB = 16
_BLK = 128


def _mesh():
    return plsc.VectorSubcoreMesh(core_axis_name="c", subcore_axis_name="s")


def _cparams(**kw):
    cp = pltpu.CompilerParams()
    fields = pltpu.CompilerParams.__dataclass_fields__
    kw = {k: v for k, v in kw.items() if k in fields}
    return dataclasses.replace(cp, **kw)


def _sc_hist(col2d, npad):
    eb = col2d.shape[0] // _NTILES

    @functools.partial(
        pl.kernel,
        out_type=jax.ShapeDtypeStruct((_NTILES, npad), _F32),
        mesh=_mesh(),
        compiler_params=_cparams(needs_layout_passes=False),
        scratch_types=[
            pltpu.VMEM((eb, _BLK), jnp.int32),
            pltpu.VMEM((npad,), _F32),
        ],
    )
    def k(col_hbm, out_hbm, idx, hist):
        c = lax.axis_index("c")
        s = lax.axis_index("s")
        wid = c * _NSUB + s
        pltpu.sync_copy(col_hbm.at[pl.ds(wid * eb, eb)], idx)

        @pl.loop(0, npad, step=16)
        def _(i):
            hist[pl.ds(i, 16)] = jnp.zeros((16,), _F32)

        ones = jnp.ones((16,), _F32)

        @pl.loop(0, eb)
        def _(j):
            @pl.loop(0, _BLK, step=16)
            def _(t):
                plsc.addupdate_scatter(hist, [idx[j, pl.ds(t, 16)]], ones)

        pltpu.sync_copy(hist, out_hbm.at[wid])

    return k(col2d)


_AGG_DEPTH = 5
_DEC_DEPTH = 4


def _sc_agg(ys, row2d, col2d, zeros_h, npad, dh):
    eb = row2d.shape[0] // _NSUB
    rows = npad // _NSUB
    nd = _AGG_DEPTH

    @functools.partial(
        pl.kernel,
        out_type=jax.ShapeDtypeStruct((2, npad, dh), _F32),
        mesh=_mesh(),
        compiler_params=_cparams(use_tc_tiling_on_sc=False),
        scratch_types=(
            [pltpu.VMEM((eb, _BLK), jnp.int32),
             pltpu.VMEM((eb, _BLK), jnp.int32)]
            + [pltpu.VMEM((_BLK, dh), _F32)] * nd
            + [pltpu.VMEM_SHARED((npad, dh), _F32)]
            + [pltpu.SemaphoreType.DMA] * (2 * nd)
        ),
    )
    def k(y_hbm, r_hbm, c_hbm, z_hbm, out_hbm, ridx, cidx, *rest):
        buf = rest[:nd]
        acc = rest[nd]
        sg = rest[nd + 1:2 * nd + 1]
        ss = rest[2 * nd + 1:]
        c = lax.axis_index("c")
        s = lax.axis_index("s")
        pltpu.sync_copy(z_hbm.at[pl.ds(s * rows, rows)],
                        acc.at[pl.ds(s * rows, rows)])
        pltpu.sync_copy(r_hbm.at[pl.ds(s * eb, eb)], ridx)
        pltpu.sync_copy(c_hbm.at[pl.ds(s * eb, eb)], cidx)
        plsc.subcore_barrier()
        ysel = y_hbm.at[c]

        for b in range(nd):
            pltpu.async_copy(ysel.at[ridx.at[b]], buf[b], sg[b])

        @pl.loop(0, eb - nd, step=nd)
        def _(j):
            for b in range(nd):
                jb = j + b
                pltpu.make_async_copy(ysel.at[ridx.at[jb]], buf[b],
                                      sg[b]).wait()
                pltpu.async_copy(buf[b], acc.at[cidx.at[jb]], ss[b],
                                 add=True)
            for b in range(nd):
                jb = j + b
                pltpu.make_async_copy(buf[b], acc.at[cidx.at[jb]],
                                      ss[b]).wait()
                pltpu.async_copy(ysel.at[ridx.at[jb + nd]], buf[b], sg[b])

        for b in range(nd):
            jb = eb - nd + b
            pltpu.make_async_copy(ysel.at[ridx.at[jb]], buf[b], sg[b]).wait()
            pltpu.async_copy(buf[b], acc.at[cidx.at[jb]], ss[b], add=True)
        for b in range(nd):
            jb = eb - nd + b
            pltpu.make_async_copy(buf[b], acc.at[cidx.at[jb]], ss[b]).wait()

        plsc.subcore_barrier()
        pltpu.sync_copy(acc.at[pl.ds(s * rows, rows)],
                        out_hbm.at[c, pl.ds(s * rows, rows)])

    return k(ys, row2d, col2d, zeros_h)


def _sc_decode(zs, src2d, dst2d, dh):
    lb = src2d.shape[0] // _NSUB
    lpad = src2d.size
    nd = _DEC_DEPTH

    @functools.partial(
        pl.kernel,
        out_type=(jax.ShapeDtypeStruct((2, lpad, dh), _F32),
                  jax.ShapeDtypeStruct((2, lpad, dh), _F32)),
        mesh=_mesh(),
        compiler_params=_cparams(use_tc_tiling_on_sc=False),
        scratch_types=(
            [pltpu.VMEM((lb, _BLK), jnp.int32),
             pltpu.VMEM((lb, _BLK), jnp.int32)]
            + [pltpu.VMEM((_BLK, dh), _F32)] * (2 * nd)
            + [pltpu.SemaphoreType.DMA] * (4 * nd)
        ),
    )
    def k(z_hbm, s_hbm, d_hbm, za_hbm, zb_hbm, sidx, didx, *rest):
        bufa = rest[:nd]
        bufb = rest[nd:2 * nd]
        sga = rest[2 * nd:3 * nd]
        sgb = rest[3 * nd:4 * nd]
        swa = rest[4 * nd:5 * nd]
        swb = rest[5 * nd:6 * nd]
        c = lax.axis_index("c")
        s = lax.axis_index("s")
        pltpu.sync_copy(s_hbm.at[pl.ds(s * lb, lb)], sidx)
        pltpu.sync_copy(d_hbm.at[pl.ds(s * lb, lb)], didx)
        zsel = z_hbm.at[c]

        def out_sl(ib):
            return pl.ds((s * lb + ib) * _BLK, _BLK)

        for b in range(nd):
            pltpu.async_copy(zsel.at[sidx.at[b]], bufa[b], sga[b])
            pltpu.async_copy(zsel.at[didx.at[b]], bufb[b], sgb[b])

        @pl.loop(0, lb - nd, step=nd)
        def _(i):
            for b in range(nd):
                ib = i + b
                pltpu.make_async_copy(zsel.at[sidx.at[ib]], bufa[b],
                                      sga[b]).wait()
                pltpu.async_copy(bufa[b], za_hbm.at[c, out_sl(ib)], swa[b])
                pltpu.make_async_copy(zsel.at[didx.at[ib]], bufb[b],
                                      sgb[b]).wait()
                pltpu.async_copy(bufb[b], zb_hbm.at[c, out_sl(ib)], swb[b])
            for b in range(nd):
                ib = i + b
                pltpu.make_async_copy(bufa[b], za_hbm.at[c, out_sl(ib)],
                                      swa[b]).wait()
                pltpu.make_async_copy(bufb[b], zb_hbm.at[c, out_sl(ib)],
                                      swb[b]).wait()
                pltpu.async_copy(zsel.at[sidx.at[ib + nd]], bufa[b], sga[b])
                pltpu.async_copy(zsel.at[didx.at[ib + nd]], bufb[b], sgb[b])

        for b in range(nd):
            ib = lb - nd + b
            pltpu.make_async_copy(zsel.at[sidx.at[ib]], bufa[b],
                                  sga[b]).wait()
            pltpu.async_copy(bufa[b], za_hbm.at[c, out_sl(ib)], swa[b])
            pltpu.make_async_copy(zsel.at[didx.at[ib]], bufb[b],
                                  sgb[b]).wait()
            pltpu.async_copy(bufb[b], zb_hbm.at[c, out_sl(ib)], swb[b])
        for b in range(nd):
            ib = lb - nd + b
            pltpu.make_async_copy(bufa[b], za_hbm.at[c, out_sl(ib)],
                                  swa[b]).wait()
            pltpu.make_async_copy(bufb[b], zb_hbm.at[c, out_sl(ib)],
                                  swb[b]).wait()

    return k(zs, src2d, dst2d)


_HIGH = jax.lax.Precision.HIGHEST


def _dot(a, b):
    return lax.dot_general(a, b, (((1,), (0,)), ((), ())),
                           precision=_HIGH, preferred_element_type=_F32)


def _dis(cnt_ref):
    deg = 1.0 + jnp.sum(cnt_ref[...], axis=1, keepdims=True)
    return 1.0 / jnp.sqrt(deg)


def _split(h, o_ref):
    dh = h.shape[1] // 2
    o_ref[0] = h[:, :dh]
    o_ref[1] = h[:, dh:]


def _tc_matmul(x, w):
    def body(x_ref, w_ref, o_ref):
        o_ref[...] = _dot(x_ref[...], w_ref[...])
    return pl.pallas_call(
        body, out_shape=jax.ShapeDtypeStruct((x.shape[0], w.shape[1]), _F32),
    )(x, w)


_ROWBLK = 2048


def _tc_prep(xw, cnt_t):
    n, d = xw.shape
    def body(xw_ref, cnt_ref, o_ref):
        _split(xw_ref[...] * _dis(cnt_ref), o_ref)
    return pl.pallas_call(
        body,
        grid=(n // _ROWBLK,),
        in_specs=[pl.BlockSpec((_ROWBLK, d), lambda i: (i, 0)),
                  pl.BlockSpec((_ROWBLK, cnt_t.shape[1]), lambda i: (i, 0))],
        out_specs=pl.BlockSpec((2, _ROWBLK, d // 2), lambda i: (0, i, 0)),
        out_shape=jax.ShapeDtypeStruct((2, n, d // 2), _F32),
    )(xw, cnt_t)


def _tc_layer_mid(ag, ys, cnt_t, b, w2):
    _, n, dh = ys.shape
    def body(a_ref, y_ref, cnt_ref, b_ref, w_ref, o_ref):
        dis = _dis(cnt_ref)
        a = jnp.concatenate([a_ref[0], a_ref[1]], axis=1)
        y = jnp.concatenate([y_ref[0], y_ref[1]], axis=1)
        z = jnp.maximum(dis * (a + y) + b_ref[...], 0.0)
        _split(_dot(z, w_ref[...]) * dis, o_ref)
    return pl.pallas_call(
        body,
        grid=(n // _ROWBLK,),
        in_specs=[pl.BlockSpec((2, _ROWBLK, dh), lambda i: (0, i, 0)),
                  pl.BlockSpec((2, _ROWBLK, dh), lambda i: (0, i, 0)),
                  pl.BlockSpec((_ROWBLK, cnt_t.shape[1]), lambda i: (i, 0)),
                  pl.BlockSpec((1, 2 * dh), lambda i: (0, 0)),
                  pl.BlockSpec((2 * dh, 2 * dh), lambda i: (0, 0))],
        out_specs=pl.BlockSpec((2, _ROWBLK, dh), lambda i: (0, i, 0)),
        out_shape=jax.ShapeDtypeStruct((2, n, dh), _F32),
    )(ag, ys, cnt_t, b, w2)


def _tc_layer_last(ag, ys, cnt_t, b):
    _, n, dh = ys.shape
    def body(a_ref, y_ref, cnt_ref, b_ref, o_ref):
        dis = _dis(cnt_ref)
        a = jnp.concatenate([a_ref[0], a_ref[1]], axis=1)
        y = jnp.concatenate([y_ref[0], y_ref[1]], axis=1)
        _split(jnp.maximum(dis * (a + y) + b_ref[...], 0.0), o_ref)
    return pl.pallas_call(
        body,
        grid=(n // _ROWBLK,),
        in_specs=[pl.BlockSpec((2, _ROWBLK, dh), lambda i: (0, i, 0)),
                  pl.BlockSpec((2, _ROWBLK, dh), lambda i: (0, i, 0)),
                  pl.BlockSpec((_ROWBLK, cnt_t.shape[1]), lambda i: (i, 0)),
                  pl.BlockSpec((1, 2 * dh), lambda i: (0, 0))],
        out_specs=pl.BlockSpec((2, _ROWBLK, dh), lambda i: (0, i, 0)),
        out_shape=jax.ShapeDtypeStruct((2, n, dh), _F32),
    )(ag, ys, cnt_t, b)


def _tc_dotred(a0, b0, a1, b1):
    blk = 4096
    dh = a0.shape[1]
    def body(a0_ref, b0_ref, a1_ref, b1_ref, o_ref):
        o_ref[...] = (
            jnp.sum(a0_ref[...] * b0_ref[...], axis=1, keepdims=True)
            + jnp.sum(a1_ref[...] * b1_ref[...], axis=1, keepdims=True))
    spec = pl.BlockSpec((blk, dh), lambda i: (i, 0))
    return pl.pallas_call(
        body,
        grid=(a0.shape[0] // blk,),
        in_specs=[spec] * 4,
        out_specs=pl.BlockSpec((blk, 1), lambda i: (i, 0)),
        out_shape=jax.ShapeDtypeStruct((a0.shape[0], 1), _F32),
    )(a0, b0, a1, b1)


def _pad_idx(a, pad_val, unit):
    n = a.shape[0]
    npad = -(-n // unit) * unit
    return jnp.concatenate(
        [a, jnp.full((npad - n,), pad_val, jnp.int32)]).reshape(-1, _BLK)


def kernel(x, edge_index, edge_label_index, W1, b1, W2, b2):
    n, d = x.shape
    dh = d // 2
    npad = -(-(n + 16) // 2048) * 2048

    row = edge_index[0].astype(jnp.int32)
    col = edge_index[1].astype(jnp.int32)
    src = edge_label_index[0].astype(jnp.int32)
    dst = edge_label_index[1].astype(jnp.int32)
    nlabel = src.shape[0]

    unit = _NTILES * _BLK * 8
    row2d = _pad_idx(row, n, unit)
    col2d = _pad_idx(col, n + 8, unit)
    src2d = _pad_idx(src, 0, unit)
    dst2d = _pad_idx(dst, 0, unit)

    x_p = jnp.pad(x, ((0, npad - n), (0, 0)))
    zeros_h = jnp.zeros((npad, dh), _F32)
    b1r = b1.reshape(1, d)
    b2r = b2.reshape(1, d)

    cnt = _sc_hist(col2d, npad)
    xw1 = _tc_matmul(x_p, W1)
    cnt_t = cnt.T

    ys = _tc_prep(xw1, cnt_t)
    ag1 = _sc_agg(ys, row2d, col2d, zeros_h, npad, dh)
    y2s = _tc_layer_mid(ag1, ys, cnt_t, b1r, W2)
    ag2 = _sc_agg(y2s, row2d, col2d, zeros_h, npad, dh)
    zs = _tc_layer_last(ag2, y2s, cnt_t, b2r)

    za, zb = _sc_decode(zs, src2d, dst2d, dh)
    scores = _tc_dotred(za[0], zb[0], za[1], zb[1])
    return scores[:nlabel, 0]


# speedup vs baseline: 5.3023x; 1.6148x over previous
import dataclasses
import functools

import jax
import jax.numpy as jnp
from jax import lax
from jax.experimental import pallas as pl
from jax.experimental.pallas import tpu as pltpu
from jax.experimental.pallas import tpu_sc as plsc

_F32 = jnp.float32
_NTILES = 32
_NSUB = 16
_BLK = 128


def _mesh():
    return plsc.VectorSubcoreMesh(core_axis_name="c", subcore_axis_name="s")


def _cparams(**kw):
    cp = pltpu.CompilerParams()
    fields = pltpu.CompilerParams.__dataclass_fields__
    kw = {k: v for k, v in kw.items() if k in fields}
    return dataclasses.replace(cp, **kw)


def _sc_hist(col2d, npad):
    eb = col2d.shape[0] // _NTILES

    @functools.partial(
        pl.kernel,
        out_type=jax.ShapeDtypeStruct((_NTILES, npad), _F32),
        mesh=_mesh(),
        compiler_params=_cparams(needs_layout_passes=False),
        scratch_types=[
            pltpu.VMEM((eb, _BLK), jnp.int32),
            pltpu.VMEM((npad,), _F32),
        ],
    )
    def k(col_hbm, out_hbm, idx, hist):
        c = lax.axis_index("c")
        s = lax.axis_index("s")
        wid = c * _NSUB + s
        pltpu.sync_copy(col_hbm.at[pl.ds(wid * eb, eb)], idx)

        @pl.loop(0, npad, step=16)
        def _(i):
            hist[pl.ds(i, 16)] = jnp.zeros((16,), _F32)

        ones = jnp.ones((16,), _F32)

        @pl.loop(0, eb)
        def _(j):
            @pl.loop(0, _BLK, step=16)
            def _(t):
                plsc.addupdate_scatter(hist, [idx[j, pl.ds(t, 16)]], ones)

        pltpu.sync_copy(hist, out_hbm.at[wid])

    return k(col2d)


_AGG_DEPTH = 5
_DEC_DEPTH = 4


def _sc_agg(ys, row2d, col2d, zeros_h, npad, dh):
    eb = row2d.shape[0] // _NSUB
    rows = npad // _NSUB
    nd = _AGG_DEPTH

    @functools.partial(
        pl.kernel,
        out_type=jax.ShapeDtypeStruct((2, npad, dh), _F32),
        mesh=_mesh(),
        compiler_params=_cparams(use_tc_tiling_on_sc=False),
        scratch_types=(
            [pltpu.VMEM((eb, _BLK), jnp.int32),
             pltpu.VMEM((eb, _BLK), jnp.int32)]
            + [pltpu.VMEM((_BLK, dh), _F32)] * nd
            + [pltpu.VMEM_SHARED((npad, dh), _F32)]
            + [pltpu.SemaphoreType.DMA] * (2 * nd)
        ),
    )
    def k(y_hbm, r_hbm, c_hbm, z_hbm, out_hbm, ridx, cidx, *rest):
        buf = rest[:nd]
        acc = rest[nd]
        sg = rest[nd + 1:2 * nd + 1]
        ss = rest[2 * nd + 1:]
        c = lax.axis_index("c")
        s = lax.axis_index("s")
        pltpu.sync_copy(z_hbm.at[pl.ds(s * rows, rows)],
                        acc.at[pl.ds(s * rows, rows)])
        pltpu.sync_copy(r_hbm.at[pl.ds(s * eb, eb)], ridx)
        pltpu.sync_copy(c_hbm.at[pl.ds(s * eb, eb)], cidx)
        plsc.subcore_barrier()
        ysel = y_hbm.at[c]

        for b in range(nd):
            pltpu.async_copy(ysel.at[ridx.at[b]], buf[b], sg[b])

        @pl.loop(0, eb - nd, step=nd)
        def _(j):
            for b in range(nd):
                jb = j + b
                pltpu.make_async_copy(ysel.at[ridx.at[jb]], buf[b],
                                      sg[b]).wait()
                pltpu.async_copy(buf[b], acc.at[cidx.at[jb]], ss[b],
                                 add=True)
            for b in range(nd):
                jb = j + b
                pltpu.make_async_copy(buf[b], acc.at[cidx.at[jb]],
                                      ss[b]).wait()
                pltpu.async_copy(ysel.at[ridx.at[jb + nd]], buf[b], sg[b])

        for b in range(nd):
            jb = eb - nd + b
            pltpu.make_async_copy(ysel.at[ridx.at[jb]], buf[b], sg[b]).wait()
            pltpu.async_copy(buf[b], acc.at[cidx.at[jb]], ss[b], add=True)
        for b in range(nd):
            jb = eb - nd + b
            pltpu.make_async_copy(buf[b], acc.at[cidx.at[jb]], ss[b]).wait()

        plsc.subcore_barrier()
        pltpu.sync_copy(acc.at[pl.ds(s * rows, rows)],
                        out_hbm.at[c, pl.ds(s * rows, rows)])

    return k(ys, row2d, col2d, zeros_h)


def _sc_decode(z, src2d, dst2d):
    d = z.shape[1]
    eblk = 64
    lb = src2d.shape[0] // _NTILES
    lpad = src2d.size
    nd = _DEC_DEPTH

    @functools.partial(
        pl.kernel,
        out_type=(jax.ShapeDtypeStruct((lpad, d), _F32),
                  jax.ShapeDtypeStruct((lpad, d), _F32)),
        mesh=_mesh(),
        compiler_params=_cparams(use_tc_tiling_on_sc=False),
        scratch_types=(
            [pltpu.VMEM((lb, eblk), jnp.int32),
             pltpu.VMEM((lb, eblk), jnp.int32)]
            + [pltpu.VMEM((eblk, d), _F32)] * (2 * nd)
            + [pltpu.SemaphoreType.DMA] * (4 * nd)
        ),
    )
    def k(z_hbm, s_hbm, d_hbm, za_hbm, zb_hbm, sidx, didx, *rest):
        bufa = rest[:nd]
        bufb = rest[nd:2 * nd]
        sga = rest[2 * nd:3 * nd]
        sgb = rest[3 * nd:4 * nd]
        swa = rest[4 * nd:5 * nd]
        swb = rest[5 * nd:6 * nd]
        c = lax.axis_index("c")
        s = lax.axis_index("s")
        wid = c * _NSUB + s
        pltpu.sync_copy(s_hbm.at[pl.ds(wid * lb, lb)], sidx)
        pltpu.sync_copy(d_hbm.at[pl.ds(wid * lb, lb)], didx)

        def out_sl(ib):
            return pl.ds((wid * lb + ib) * eblk, eblk)

        for b in range(nd):
            pltpu.async_copy(z_hbm.at[sidx.at[b]], bufa[b], sga[b])
            pltpu.async_copy(z_hbm.at[didx.at[b]], bufb[b], sgb[b])

        @pl.loop(0, lb - nd, step=nd)
        def _(i):
            for b in range(nd):
                ib = i + b
                pltpu.make_async_copy(z_hbm.at[sidx.at[ib]], bufa[b],
                                      sga[b]).wait()
                pltpu.async_copy(bufa[b], za_hbm.at[out_sl(ib)], swa[b])
                pltpu.make_async_copy(z_hbm.at[didx.at[ib]], bufb[b],
                                      sgb[b]).wait()
                pltpu.async_copy(bufb[b], zb_hbm.at[out_sl(ib)], swb[b])
            for b in range(nd):
                ib = i + b
                pltpu.make_async_copy(bufa[b], za_hbm.at[out_sl(ib)],
                                      swa[b]).wait()
                pltpu.make_async_copy(bufb[b], zb_hbm.at[out_sl(ib)],
                                      swb[b]).wait()
                pltpu.async_copy(z_hbm.at[sidx.at[ib + nd]], bufa[b], sga[b])
                pltpu.async_copy(z_hbm.at[didx.at[ib + nd]], bufb[b], sgb[b])

        for b in range(nd):
            ib = lb - nd + b
            pltpu.make_async_copy(z_hbm.at[sidx.at[ib]], bufa[b],
                                  sga[b]).wait()
            pltpu.async_copy(bufa[b], za_hbm.at[out_sl(ib)], swa[b])
            pltpu.make_async_copy(z_hbm.at[didx.at[ib]], bufb[b],
                                  sgb[b]).wait()
            pltpu.async_copy(bufb[b], zb_hbm.at[out_sl(ib)], swb[b])
        for b in range(nd):
            ib = lb - nd + b
            pltpu.make_async_copy(bufa[b], za_hbm.at[out_sl(ib)],
                                  swa[b]).wait()
            pltpu.make_async_copy(bufb[b], zb_hbm.at[out_sl(ib)],
                                  swb[b]).wait()

    return k(z, src2d, dst2d)


_HIGH = jax.lax.Precision.HIGHEST


def _dot(a, b):
    return lax.dot_general(a, b, (((1,), (0,)), ((), ())),
                           precision=_HIGH, preferred_element_type=_F32)


def _dis(cnt_ref):
    deg = 1.0 + jnp.sum(cnt_ref[...], axis=1, keepdims=True)
    return 1.0 / jnp.sqrt(deg)


def _split(h, o_ref):
    dh = h.shape[1] // 2
    o_ref[0] = h[:, :dh]
    o_ref[1] = h[:, dh:]


def _tc_matmul(x, w):
    def body(x_ref, w_ref, o_ref):
        o_ref[...] = _dot(x_ref[...], w_ref[...])
    return pl.pallas_call(
        body, out_shape=jax.ShapeDtypeStruct((x.shape[0], w.shape[1]), _F32),
    )(x, w)


_ROWBLK = 2048


def _tc_prep(xw, cnt_t):
    n, d = xw.shape
    def body(xw_ref, cnt_ref, o_ref):
        _split(xw_ref[...] * _dis(cnt_ref), o_ref)
    return pl.pallas_call(
        body,
        grid=(n // _ROWBLK,),
        in_specs=[pl.BlockSpec((_ROWBLK, d), lambda i: (i, 0)),
                  pl.BlockSpec((_ROWBLK, cnt_t.shape[1]), lambda i: (i, 0))],
        out_specs=pl.BlockSpec((2, _ROWBLK, d // 2), lambda i: (0, i, 0)),
        out_shape=jax.ShapeDtypeStruct((2, n, d // 2), _F32),
    )(xw, cnt_t)


def _tc_layer_mid(ag, ys, cnt_t, b, w2):
    _, n, dh = ys.shape
    def body(a_ref, y_ref, cnt_ref, b_ref, w_ref, o_ref):
        dis = _dis(cnt_ref)
        a = jnp.concatenate([a_ref[0], a_ref[1]], axis=1)
        y = jnp.concatenate([y_ref[0], y_ref[1]], axis=1)
        z = jnp.maximum(dis * (a + y) + b_ref[...], 0.0)
        _split(_dot(z, w_ref[...]) * dis, o_ref)
    return pl.pallas_call(
        body,
        grid=(n // _ROWBLK,),
        in_specs=[pl.BlockSpec((2, _ROWBLK, dh), lambda i: (0, i, 0)),
                  pl.BlockSpec((2, _ROWBLK, dh), lambda i: (0, i, 0)),
                  pl.BlockSpec((_ROWBLK, cnt_t.shape[1]), lambda i: (i, 0)),
                  pl.BlockSpec((1, 2 * dh), lambda i: (0, 0)),
                  pl.BlockSpec((2 * dh, 2 * dh), lambda i: (0, 0))],
        out_specs=pl.BlockSpec((2, _ROWBLK, dh), lambda i: (0, i, 0)),
        out_shape=jax.ShapeDtypeStruct((2, n, dh), _F32),
    )(ag, ys, cnt_t, b, w2)


def _tc_layer_last(ag, ys, cnt_t, b):
    _, n, dh = ys.shape
    def body(a_ref, y_ref, cnt_ref, b_ref, o_ref):
        dis = _dis(cnt_ref)
        a = jnp.concatenate([a_ref[0], a_ref[1]], axis=1)
        y = jnp.concatenate([y_ref[0], y_ref[1]], axis=1)
        o_ref[...] = jnp.maximum(dis * (a + y) + b_ref[...], 0.0)
    return pl.pallas_call(
        body,
        grid=(n // _ROWBLK,),
        in_specs=[pl.BlockSpec((2, _ROWBLK, dh), lambda i: (0, i, 0)),
                  pl.BlockSpec((2, _ROWBLK, dh), lambda i: (0, i, 0)),
                  pl.BlockSpec((_ROWBLK, cnt_t.shape[1]), lambda i: (i, 0)),
                  pl.BlockSpec((1, 2 * dh), lambda i: (0, 0))],
        out_specs=pl.BlockSpec((_ROWBLK, 2 * dh), lambda i: (i, 0)),
        out_shape=jax.ShapeDtypeStruct((n, 2 * dh), _F32),
    )(ag, ys, cnt_t, b)


def _tc_dotred(za, zb):
    blk = 4096
    d = za.shape[1]
    def body(a_ref, b_ref, o_ref):
        o_ref[...] = jnp.sum(a_ref[...] * b_ref[...], axis=1, keepdims=True)
    spec = pl.BlockSpec((blk, d), lambda i: (i, 0))
    return pl.pallas_call(
        body,
        grid=(za.shape[0] // blk,),
        in_specs=[spec, spec],
        out_specs=pl.BlockSpec((blk, 1), lambda i: (i, 0)),
        out_shape=jax.ShapeDtypeStruct((za.shape[0], 1), _F32),
    )(za, zb)


def _pad_idx(a, pad_val, unit, blk=_BLK):
    n = a.shape[0]
    npad = -(-n // unit) * unit
    return jnp.concatenate(
        [a, jnp.full((npad - n,), pad_val, jnp.int32)]).reshape(-1, blk)


def kernel(x, edge_index, edge_label_index, W1, b1, W2, b2):
    n, d = x.shape
    dh = d // 2
    npad = -(-(n + 16) // 2048) * 2048

    row = edge_index[0].astype(jnp.int32)
    col = edge_index[1].astype(jnp.int32)
    src = edge_label_index[0].astype(jnp.int32)
    dst = edge_label_index[1].astype(jnp.int32)
    nlabel = src.shape[0]

    unit = _NTILES * _BLK * 8
    row2d = _pad_idx(row, n, unit)
    col2d = _pad_idx(col, n + 8, unit)
    lunit = _NTILES * 64 * 8
    src2d = _pad_idx(src, 0, lunit, blk=64)
    dst2d = _pad_idx(dst, 0, lunit, blk=64)

    x_p = jnp.pad(x, ((0, npad - n), (0, 0)))
    zeros_h = jnp.zeros((npad, dh), _F32)
    b1r = b1.reshape(1, d)
    b2r = b2.reshape(1, d)

    cnt = _sc_hist(col2d, npad)
    xw1 = _tc_matmul(x_p, W1)
    cnt_t = cnt.T

    ys = _tc_prep(xw1, cnt_t)
    ag1 = _sc_agg(ys, row2d, col2d, zeros_h, npad, dh)
    y2s = _tc_layer_mid(ag1, ys, cnt_t, b1r, W2)
    ag2 = _sc_agg(y2s, row2d, col2d, zeros_h, npad, dh)
    z2 = _tc_layer_last(ag2, y2s, cnt_t, b2r)

    za, zb = _sc_decode(z2, src2d, dst2d)
    scores = _tc_dotred(za, zb)
    return scores[:nlabel, 0]
